# Initial kernel scaffold; baseline (speedup 1.0000x reference)
import functools

import jax
import jax.numpy as jnp
import numpy as np
from jax import lax
from jax.experimental import pallas as pl
from jax.experimental.pallas import tpu as pltpu
from jax.experimental.pallas import tpu_sc as plsc

N = 50000
E = 800000
DIM = 64
HALF = DIM // 2
CUTOFF = 5.0
WIDTH = 0.1
N_CONV = 3
N_CENTERS = 50
_CENTERS = np.linspace(0.0, CUTOFF, N_CENTERS).astype(np.float32)
GAP = float(_CENTERS[1] - _CENTERS[0])

NCORES = 2
NSUB = 16
CH = 1024
SUBOP = 128
NSUBOP = CH // SUBOP
EPAD = 819200
EPT = EPAD // NSUB
NCHUNK = EPT // CH
NACC = 52000
ROWS_PT = NACC // NSUB

NB = 2000
NBLKN = N // NB
EB = 4096
NEB = EPAD // EB
ACC_BLK_OFF = NACC // NB


def _sp05(x):
    return jnp.where(0.5 * x > 14.0, x,
                     2.0 * jnp.log1p(jnp.exp(jnp.minimum(0.5 * x, 14.0))))


def _ssp(x):
    sp = jnp.where(x > 20.0, x, jnp.log1p(jnp.exp(jnp.minimum(x, 20.0))))
    return sp - np.log(2.0)


def _emb_body(nt_ref, emb_ref, w1h_ref, node_ref, nn_ref):
    nt = nt_ref[0, 0, :]
    oh = (nt[:, None] == lax.broadcasted_iota(jnp.int32, (NB, 128), 1)
          ).astype(jnp.float32)
    node = jax.lax.dot_general(oh, emb_ref[...], (((1,), (0,)), ((), ())),
                               preferred_element_type=jnp.float32)
    node_ref[...] = node
    nn_ref[...] = jnp.dot(node, w1h_ref[...],
                          preferred_element_type=jnp.float32)


def _emb_call(nt3, emb128, w1):
    return pl.pallas_call(
        _emb_body,
        grid=(NBLKN, NCORES),
        in_specs=[
            pl.BlockSpec((1, 1, NB), lambda i, j: (i, 0, 0)),
            pl.BlockSpec((128, DIM), lambda i, j: (0, 0)),
            pl.BlockSpec((DIM, HALF), lambda i, j: (0, j)),
        ],
        out_specs=[
            pl.BlockSpec((NB, DIM), lambda i, j: (i, 0)),
            pl.BlockSpec((NB, HALF), lambda i, j: (j * NBLKN + i, 0)),
        ],
        out_shape=[
            jax.ShapeDtypeStruct((N, DIM), jnp.float32),
            jax.ShapeDtypeStruct((NCORES * N, HALF), jnp.float32),
        ],
    )(nt3, emb128, w1)


def _h_body(dist_ref, cent_ref, cw1a, cb1a, cw2a, cb2a,
            cw1b, cb1b, cw2b, cb2b, cw1c, cb1c, cw2c, cb2c,
            h0_ref, h1_ref, h2_ref):
    d = dist_ref[...]
    radial = d - cent_ref[...]
    rbf = jnp.exp((-1.0 / GAP) * radial * radial)
    for cw1, cb1, cw2, cb2, out in (
            (cw1a, cb1a, cw2a, cb2a, h0_ref),
            (cw1b, cb1b, cw2b, cb2b, h1_ref),
            (cw1c, cb1c, cw2c, cb2c, h2_ref)):
        t = _sp05(jnp.dot(rbf, cw1[...], preferred_element_type=jnp.float32)
                  + cb1[...])
        t = _sp05(jnp.dot(t, cw2[...], preferred_element_type=jnp.float32)
                  + cb2[...])
        out[...] = t


def _h_call(dist_pad, centers, params):
    ins = [dist_pad, centers]
    in_specs = [
        pl.BlockSpec((EB, 1), lambda i, j: (i, 0)),
        pl.BlockSpec((1, N_CENTERS), lambda i, j: (0, 0)),
    ]
    for l in range(N_CONV):
        ins += [params['cw1_%d' % l], params['cb1_%d' % l].reshape(1, DIM),
                params['cw2_%d' % l], params['cb2_%d' % l].reshape(1, DIM)]
        in_specs += [
            pl.BlockSpec((N_CENTERS, DIM), lambda i, j: (0, 0)),
            pl.BlockSpec((1, DIM), lambda i, j: (0, 0)),
            pl.BlockSpec((DIM, HALF), lambda i, j: (0, j)),
            pl.BlockSpec((1, HALF), lambda i, j: (0, j)),
        ]
    return pl.pallas_call(
        _h_body,
        grid=(NEB, NCORES),
        in_specs=in_specs,
        out_specs=[pl.BlockSpec((EB, HALF), lambda i, j: (j * NEB + i, 0))
                   for _ in range(N_CONV)],
        out_shape=[jax.ShapeDtypeStruct((NCORES * EPAD, HALF), jnp.float32)
                   for _ in range(N_CONV)],
    )(*ins)


def _sc_body(nn_ref, h_ref, src_ref, dst_ref, agg_ref,
             sbuf, dbuf, rbuf, hbuf, acc, gsem, hsem):
    c = lax.axis_index("c")
    s = lax.axis_index("s")

    def zbody(i, _):
        rbuf[i, pl.ds(0, 16)] = jnp.zeros((16,), jnp.float32)
        rbuf[i, pl.ds(16, 16)] = jnp.zeros((16,), jnp.float32)
        return 0
    lax.fori_loop(0, CH, zbody, 0)
    r0 = s * ROWS_PT
    for m in range(ROWS_PT // CH):
        pltpu.sync_copy(rbuf, acc.at[pl.ds(r0 + m * CH, CH)])
    rem = ROWS_PT % CH
    if rem:
        pltpu.sync_copy(rbuf.at[pl.ds(0, rem)],
                        acc.at[pl.ds(r0 + (ROWS_PT // CH) * CH, rem)])
    plsc.subcore_barrier()

    ebase = s * EPT
    ibase = c * (EPAD // SUBOP) + s * (EPT // SUBOP)

    def chunk(k, _):
        pltpu.sync_copy(src_ref.at[pl.ds(ibase + k * NSUBOP, NSUBOP)], sbuf)
        pltpu.sync_copy(dst_ref.at[pl.ds(s * (EPT // SUBOP) + k * NSUBOP,
                                         NSUBOP)], dbuf)
        hcp = pltpu.async_copy(
            h_ref.at[pl.ds(c * EPAD + ebase + k * CH, CH)], hbuf, hsem)
        gcps = [pltpu.async_copy(nn_ref.at[sbuf.at[j]],
                                 rbuf.at[pl.ds(j * SUBOP, SUBOP)], gsem)
                for j in range(NSUBOP)]
        hcp.wait()
        for cp in gcps:
            cp.wait()

        def mbody(i, _):
            rbuf[i, pl.ds(0, 16)] = rbuf[i, pl.ds(0, 16)] * hbuf[i, pl.ds(0, 16)]
            rbuf[i, pl.ds(16, 16)] = (rbuf[i, pl.ds(16, 16)]
                                      * hbuf[i, pl.ds(16, 16)])
            return 0
        lax.fori_loop(0, CH, mbody, 0)

        scps = [pltpu.async_copy(rbuf.at[pl.ds(j * SUBOP, SUBOP)],
                                 acc.at[dbuf.at[j]], gsem, add=True)
                for j in range(NSUBOP)]
        for cp in scps:
            cp.wait()
        return 0

    lax.fori_loop(0, NCHUNK, chunk, 0)
    plsc.subcore_barrier()

    pltpu.sync_copy(acc.at[pl.ds(r0, ROWS_PT)],
                    agg_ref.at[pl.ds(c * NACC + r0, ROWS_PT)])


def _sc_call(nn2, h2, src2d, dst2d):
    mesh = plsc.VectorSubcoreMesh(core_axis_name="c", subcore_axis_name="s",
                                  num_cores=NCORES, num_subcores=NSUB)
    return pl.kernel(
        _sc_body,
        out_type=jax.ShapeDtypeStruct((NCORES * NACC, HALF), jnp.float32),
        mesh=mesh,
        scratch_types=[
            pltpu.VMEM((NSUBOP, SUBOP), jnp.int32),
            pltpu.VMEM((NSUBOP, SUBOP), jnp.int32),
            pltpu.VMEM((CH, HALF), jnp.float32),
            pltpu.VMEM((CH, HALF), jnp.float32),
            pltpu.VMEM_SHARED((NACC, HALF), jnp.float32),
            pltpu.SemaphoreType.DMA,
            pltpu.SemaphoreType.DMA,
        ],
    )(nn2, h2, src2d, dst2d)


def _upd_body(node_ref, agglo_ref, agghi_ref, w2_ref, b2_ref, w3_ref, b3_ref,
              w1h_ref, node_out_ref, nn_ref):
    w2 = w2_ref[...]
    pre = (jnp.dot(agglo_ref[...], w2[:HALF, :],
                   preferred_element_type=jnp.float32)
           + jnp.dot(agghi_ref[...], w2[HALF:, :],
                     preferred_element_type=jnp.float32)
           + b2_ref[...])
    a = _sp05(pre)
    node = (node_ref[...]
            + jnp.dot(a, w3_ref[...], preferred_element_type=jnp.float32)
            + b3_ref[...])
    node_out_ref[...] = node
    nn_ref[...] = jnp.dot(node, w1h_ref[...],
                          preferred_element_type=jnp.float32)


def _upd_call(node, agg, params, l):
    return pl.pallas_call(
        _upd_body,
        grid=(NBLKN, NCORES),
        in_specs=[
            pl.BlockSpec((NB, DIM), lambda i, j: (i, 0)),
            pl.BlockSpec((NB, HALF), lambda i, j: (i, 0)),
            pl.BlockSpec((NB, HALF), lambda i, j: (ACC_BLK_OFF + i, 0)),
            pl.BlockSpec((DIM, DIM), lambda i, j: (0, 0)),
            pl.BlockSpec((1, DIM), lambda i, j: (0, 0)),
            pl.BlockSpec((DIM, DIM), lambda i, j: (0, 0)),
            pl.BlockSpec((1, DIM), lambda i, j: (0, 0)),
            pl.BlockSpec((DIM, HALF), lambda i, j: (0, j)),
        ],
        out_specs=[
            pl.BlockSpec((NB, DIM), lambda i, j: (i, 0)),
            pl.BlockSpec((NB, HALF), lambda i, j: (j * NBLKN + i, 0)),
        ],
        out_shape=[
            jax.ShapeDtypeStruct((N, DIM), jnp.float32),
            jax.ShapeDtypeStruct((NCORES * N, HALF), jnp.float32),
        ],
    )(node, agg, agg, params['w2_%d' % l], params['b2_%d' % l].reshape(1, DIM),
      params['w3_%d' % l], params['b3_%d' % l].reshape(1, DIM),
      params['w1_%d' % ((l + 1) % N_CONV)])


def _fin_body(node_ref, agglo_ref, agghi_ref, w2_ref, b2_ref, w3_ref, b3_ref,
              ad1w_ref, ad1b_ref, ad2w_ref, ad2b_ref, out_ref):
    i = pl.program_id(0)
    w2 = w2_ref[...]
    pre = (jnp.dot(agglo_ref[...], w2[:HALF, :],
                   preferred_element_type=jnp.float32)
           + jnp.dot(agghi_ref[...], w2[HALF:, :],
                     preferred_element_type=jnp.float32)
           + b2_ref[...])
    a = _sp05(pre)
    node = (node_ref[...]
            + jnp.dot(a, w3_ref[...], preferred_element_type=jnp.float32)
            + b3_ref[...])
    atom = _ssp(jnp.dot(node, ad1w_ref[...],
                        preferred_element_type=jnp.float32) + ad1b_ref[...])
    res = jnp.dot(atom, ad2w_ref[...], preferred_element_type=jnp.float32)
    part = (jnp.sum(res) + NB * ad2b_ref[0, 0]) * (1.0 / N)

    @pl.when(i == 0)
    def _():
        out_ref[0, 0] = 0.0
    out_ref[0, 0] += part


def _fin_call(node, agg, params):
    return pl.pallas_call(
        _fin_body,
        grid=(NBLKN,),
        in_specs=[
            pl.BlockSpec((NB, DIM), lambda i: (i, 0)),
            pl.BlockSpec((NB, HALF), lambda i: (i, 0)),
            pl.BlockSpec((NB, HALF), lambda i: (ACC_BLK_OFF + i, 0)),
            pl.BlockSpec((DIM, DIM), lambda i: (0, 0)),
            pl.BlockSpec((1, DIM), lambda i: (0, 0)),
            pl.BlockSpec((DIM, DIM), lambda i: (0, 0)),
            pl.BlockSpec((1, DIM), lambda i: (0, 0)),
            pl.BlockSpec((DIM, DIM), lambda i: (0, 0)),
            pl.BlockSpec((1, DIM), lambda i: (0, 0)),
            pl.BlockSpec((DIM, 1), lambda i: (0, 0)),
            pl.BlockSpec((1, 1), lambda i: (0, 0)),
        ],
        out_specs=pl.BlockSpec((1, 1), lambda i: (0, 0)),
        out_shape=jax.ShapeDtypeStruct((1, 1), jnp.float32),
    )(node, agg, agg, params['w2_2'], params['b2_2'].reshape(1, DIM),
      params['w3_2'], params['b3_2'].reshape(1, DIM),
      params['ad1_w'], params['ad1_b'].reshape(1, DIM),
      params['ad2_w'], params['ad2_b'].reshape(1, 1))


def kernel(node_type, edge_index, dist, emb, params):
    nt3 = node_type.astype(jnp.int32).reshape(NBLKN, 1, NB)
    emb128 = jnp.pad(emb, ((0, 128 - emb.shape[0]), (0, 0)))
    src = edge_index[0].astype(jnp.int32)
    dst = edge_index[1].astype(jnp.int32)
    src_pad = jnp.pad(src, (0, EPAD - E))
    dst_pad = jnp.pad(dst, (0, EPAD - E), constant_values=N)
    src2d = jnp.concatenate([src_pad, src_pad + N]).reshape(-1, SUBOP)
    dst2d = dst_pad.reshape(-1, SUBOP)
    dist_pad = jnp.pad(dist, ((0, EPAD - E), (0, 0)))
    centers = jnp.asarray(_CENTERS).reshape(1, N_CENTERS)

    node, nn2 = _emb_call(nt3, emb128, params['w1_0'])
    hs = _h_call(dist_pad, centers, params)
    agg = None
    for l in range(N_CONV):
        agg = _sc_call(nn2, hs[l], src2d, dst2d)
        if l < N_CONV - 1:
            node, nn2 = _upd_call(node, agg, params, l)
    return _fin_call(node, agg, params)


# speedup vs baseline: 1.5193x; 1.5193x over previous
import functools

import jax
import jax.numpy as jnp
import numpy as np
from jax import lax
from jax.experimental import pallas as pl
from jax.experimental.pallas import tpu as pltpu
from jax.experimental.pallas import tpu_sc as plsc

N = 50000
E = 800000
DIM = 64
HALF = DIM // 2
CUTOFF = 5.0
WIDTH = 0.1
N_CONV = 3
N_CENTERS = 50
_CENTERS = np.linspace(0.0, CUTOFF, N_CENTERS).astype(np.float32)
GAP = float(_CENTERS[1] - _CENTERS[0])

NCORES = 2
NSUB = 16
CH = 256
SUBOP = 128
NSUBOP = CH // SUBOP
EPAD = 819200
EPT = EPAD // NSUB
NCHUNK = EPT // CH
NACC = 50048
ROWS_PT = NACC // NSUB

NB = 2000
NBLKN = N // NB
EB = 4096
NEB = EPAD // EB


def _sp05(x):
    return jnp.where(0.5 * x > 14.0, x,
                     2.0 * jnp.log1p(jnp.exp(jnp.minimum(0.5 * x, 14.0))))


def _ssp(x):
    sp = jnp.where(x > 20.0, x, jnp.log1p(jnp.exp(jnp.minimum(x, 20.0))))
    return sp - np.log(2.0)


def _emb_body(nt_ref, emb_ref, w1h_ref, node_ref, nn_ref):
    nt = nt_ref[0, 0, :]
    oh = (nt[:, None] == lax.broadcasted_iota(jnp.int32, (NB, 128), 1)
          ).astype(jnp.float32)
    node = jax.lax.dot_general(oh, emb_ref[...], (((1,), (0,)), ((), ())),
                               preferred_element_type=jnp.float32)
    node_ref[...] = node
    nn_ref[...] = jnp.dot(node, w1h_ref[0],
                          preferred_element_type=jnp.float32)


def _emb_call(nt3, emb128, w1s):
    return pl.pallas_call(
        _emb_body,
        grid=(NBLKN, NCORES),
        in_specs=[
            pl.BlockSpec((1, 1, NB), lambda i, j: (i, 0, 0)),
            pl.BlockSpec((128, DIM), lambda i, j: (0, 0)),
            pl.BlockSpec((1, DIM, HALF), lambda i, j: (j, 0, 0)),
        ],
        out_specs=[
            pl.BlockSpec((NB, DIM), lambda i, j: (i, 0)),
            pl.BlockSpec((NB, HALF), lambda i, j: (j * NBLKN + i, 0)),
        ],
        out_shape=[
            jax.ShapeDtypeStruct((N, DIM), jnp.float32),
            jax.ShapeDtypeStruct((NCORES * N, HALF), jnp.float32),
        ],
    )(nt3, emb128, w1s)


def _h_body(dist_ref, cent_ref, cw1a, cb1a, cw2a, cb2a,
            cw1b, cb1b, cw2b, cb2b, cw1c, cb1c, cw2c, cb2c,
            h0_ref, h1_ref, h2_ref):
    d = dist_ref[...]
    radial = d - cent_ref[...]
    rbf = jnp.exp((-1.0 / GAP) * radial * radial)
    for cw1, cb1, cw2, cb2, out in (
            (cw1a, cb1a, cw2a, cb2a, h0_ref),
            (cw1b, cb1b, cw2b, cb2b, h1_ref),
            (cw1c, cb1c, cw2c, cb2c, h2_ref)):
        t = _sp05(jnp.dot(rbf, cw1[...], preferred_element_type=jnp.float32)
                  + cb1[...])
        t = _sp05(jnp.dot(t, cw2[0], preferred_element_type=jnp.float32)
                  + cb2[0])
        out[...] = t


def _h_call(dist_pad, centers, params):
    ins = [dist_pad, centers]
    in_specs = [
        pl.BlockSpec((EB, 1), lambda i, j: (i, 0)),
        pl.BlockSpec((1, N_CENTERS), lambda i, j: (0, 0)),
    ]
    for l in range(N_CONV):
        cw2 = params['cw2_%d' % l]
        cb2 = params['cb2_%d' % l]
        ins += [params['cw1_%d' % l], params['cb1_%d' % l].reshape(1, DIM),
                jnp.stack([cw2[:, :HALF], cw2[:, HALF:]]),
                jnp.stack([cb2[:HALF], cb2[HALF:]])[:, None, :]]
        in_specs += [
            pl.BlockSpec((N_CENTERS, DIM), lambda i, j: (0, 0)),
            pl.BlockSpec((1, DIM), lambda i, j: (0, 0)),
            pl.BlockSpec((1, DIM, HALF), lambda i, j: (j, 0, 0)),
            pl.BlockSpec((1, 1, HALF), lambda i, j: (j, 0, 0)),
        ]
    return pl.pallas_call(
        _h_body,
        grid=(NEB, NCORES),
        in_specs=in_specs,
        out_specs=[pl.BlockSpec((EB, HALF), lambda i, j: (j * NEB + i, 0))
                   for _ in range(N_CONV)],
        out_shape=[jax.ShapeDtypeStruct((NCORES * EPAD, HALF), jnp.float32)
                   for _ in range(N_CONV)],
    )(*ins)


def _sc_body(nn_ref, h_ref, src_ref, dst_ref, agg_ref,
             sbuf, dbuf, rbuf, hbuf, acc, gsem, hsem):
    c = lax.axis_index("c")
    s = lax.axis_index("s")

    def zbody(i, _):
        rbuf[i, pl.ds(0, 16)] = jnp.zeros((16,), jnp.float32)
        rbuf[i, pl.ds(16, 16)] = jnp.zeros((16,), jnp.float32)
        return 0
    lax.fori_loop(0, CH, zbody, 0)
    r0 = s * ROWS_PT
    for m in range(ROWS_PT // CH):
        pltpu.sync_copy(rbuf, acc.at[pl.ds(r0 + m * CH, CH)])
    rem = ROWS_PT % CH
    if rem:
        pltpu.sync_copy(rbuf.at[pl.ds(0, rem)],
                        acc.at[pl.ds(r0 + (ROWS_PT // CH) * CH, rem)])
    plsc.subcore_barrier()

    ebase = s * EPT
    ibase = c * (EPAD // SUBOP) + s * (EPT // SUBOP)

    def chunk(k, _):
        pltpu.sync_copy(src_ref.at[pl.ds(ibase + k * NSUBOP, NSUBOP)], sbuf)
        pltpu.sync_copy(dst_ref.at[pl.ds(s * (EPT // SUBOP) + k * NSUBOP,
                                         NSUBOP)], dbuf)
        hcp = pltpu.async_copy(
            h_ref.at[pl.ds(c * EPAD + ebase + k * CH, CH)], hbuf, hsem)
        gcps = [pltpu.async_copy(nn_ref.at[sbuf.at[j]],
                                 rbuf.at[pl.ds(j * SUBOP, SUBOP)], gsem)
                for j in range(NSUBOP)]
        hcp.wait()
        for cp in gcps:
            cp.wait()

        def mbody(i, _):
            rbuf[i, pl.ds(0, 16)] = rbuf[i, pl.ds(0, 16)] * hbuf[i, pl.ds(0, 16)]
            rbuf[i, pl.ds(16, 16)] = (rbuf[i, pl.ds(16, 16)]
                                      * hbuf[i, pl.ds(16, 16)])
            return 0
        lax.fori_loop(0, CH, mbody, 0)

        scps = [pltpu.async_copy(rbuf.at[pl.ds(j * SUBOP, SUBOP)],
                                 acc.at[dbuf.at[j]], gsem, add=True)
                for j in range(NSUBOP)]
        for cp in scps:
            cp.wait()
        return 0

    lax.fori_loop(0, NCHUNK, chunk, 0)
    plsc.subcore_barrier()

    pltpu.sync_copy(acc.at[pl.ds(r0, ROWS_PT)],
                    agg_ref.at[c, pl.ds(r0, ROWS_PT)])


def _sc_call(nn2, h2, src2d, dst2d):
    mesh = plsc.VectorSubcoreMesh(core_axis_name="c", subcore_axis_name="s",
                                  num_cores=NCORES, num_subcores=NSUB)
    return pl.kernel(
        _sc_body,
        out_type=jax.ShapeDtypeStruct((NCORES, NACC, HALF), jnp.float32),
        mesh=mesh,
        compiler_params=pltpu.CompilerParams(use_tc_tiling_on_sc=False),
        scratch_types=[
            pltpu.VMEM((NSUBOP, SUBOP), jnp.int32),
            pltpu.VMEM((NSUBOP, SUBOP), jnp.int32),
            pltpu.VMEM((CH, HALF), jnp.float32),
            pltpu.VMEM((CH, HALF), jnp.float32),
            pltpu.VMEM_SHARED((NACC, HALF), jnp.float32),
            pltpu.SemaphoreType.DMA,
            pltpu.SemaphoreType.DMA,
        ],
    )(nn2, h2, src2d, dst2d)


def _upd_body(node_ref, agglo_ref, agghi_ref, w2_ref, b2_ref, w3_ref, b3_ref,
              w1h_ref, node_out_ref, nn_ref):
    w2 = w2_ref[...]
    pre = (jnp.dot(agglo_ref[0], w2[:HALF, :],
                   preferred_element_type=jnp.float32)
           + jnp.dot(agghi_ref[0], w2[HALF:, :],
                     preferred_element_type=jnp.float32)
           + b2_ref[...])
    a = _sp05(pre)
    node = (node_ref[...]
            + jnp.dot(a, w3_ref[...], preferred_element_type=jnp.float32)
            + b3_ref[...])
    node_out_ref[...] = node
    nn_ref[...] = jnp.dot(node, w1h_ref[0],
                          preferred_element_type=jnp.float32)


def _upd_call(node, agg, params, w1s, l):
    return pl.pallas_call(
        _upd_body,
        grid=(NBLKN, NCORES),
        in_specs=[
            pl.BlockSpec((NB, DIM), lambda i, j: (i, 0)),
            pl.BlockSpec((1, NB, HALF), lambda i, j: (0, i, 0)),
            pl.BlockSpec((1, NB, HALF), lambda i, j: (1, i, 0)),
            pl.BlockSpec((DIM, DIM), lambda i, j: (0, 0)),
            pl.BlockSpec((1, DIM), lambda i, j: (0, 0)),
            pl.BlockSpec((DIM, DIM), lambda i, j: (0, 0)),
            pl.BlockSpec((1, DIM), lambda i, j: (0, 0)),
            pl.BlockSpec((1, DIM, HALF), lambda i, j: (j, 0, 0)),
        ],
        out_specs=[
            pl.BlockSpec((NB, DIM), lambda i, j: (i, 0)),
            pl.BlockSpec((NB, HALF), lambda i, j: (j * NBLKN + i, 0)),
        ],
        out_shape=[
            jax.ShapeDtypeStruct((N, DIM), jnp.float32),
            jax.ShapeDtypeStruct((NCORES * N, HALF), jnp.float32),
        ],
    )(node, agg, agg, params['w2_%d' % l], params['b2_%d' % l].reshape(1, DIM),
      params['w3_%d' % l], params['b3_%d' % l].reshape(1, DIM), w1s)


def _fin_body(node_ref, agglo_ref, agghi_ref, w2_ref, b2_ref, w3_ref, b3_ref,
              ad1w_ref, ad1b_ref, ad2w_ref, ad2b_ref, out_ref):
    i = pl.program_id(0)
    w2 = w2_ref[...]
    pre = (jnp.dot(agglo_ref[0], w2[:HALF, :],
                   preferred_element_type=jnp.float32)
           + jnp.dot(agghi_ref[0], w2[HALF:, :],
                     preferred_element_type=jnp.float32)
           + b2_ref[...])
    a = _sp05(pre)
    node = (node_ref[...]
            + jnp.dot(a, w3_ref[...], preferred_element_type=jnp.float32)
            + b3_ref[...])
    atom = _ssp(jnp.dot(node, ad1w_ref[...],
                        preferred_element_type=jnp.float32) + ad1b_ref[...])
    res = jnp.dot(atom, ad2w_ref[...], preferred_element_type=jnp.float32)
    part = (jnp.sum(res) + NB * ad2b_ref[0, 0]) * (1.0 / N)

    @pl.when(i == 0)
    def _():
        out_ref[...] = jnp.zeros((1, 1), jnp.float32)
    out_ref[...] = out_ref[...] + part


def _fin_call(node, agg, params):
    return pl.pallas_call(
        _fin_body,
        grid=(NBLKN,),
        in_specs=[
            pl.BlockSpec((NB, DIM), lambda i: (i, 0)),
            pl.BlockSpec((1, NB, HALF), lambda i: (0, i, 0)),
            pl.BlockSpec((1, NB, HALF), lambda i: (1, i, 0)),
            pl.BlockSpec((DIM, DIM), lambda i: (0, 0)),
            pl.BlockSpec((1, DIM), lambda i: (0, 0)),
            pl.BlockSpec((DIM, DIM), lambda i: (0, 0)),
            pl.BlockSpec((1, DIM), lambda i: (0, 0)),
            pl.BlockSpec((DIM, DIM), lambda i: (0, 0)),
            pl.BlockSpec((1, DIM), lambda i: (0, 0)),
            pl.BlockSpec((DIM, 1), lambda i: (0, 0)),
            pl.BlockSpec((1, 1), lambda i: (0, 0)),
        ],
        out_specs=pl.BlockSpec((1, 1), lambda i: (0, 0)),
        out_shape=jax.ShapeDtypeStruct((1, 1), jnp.float32),
    )(node, agg, agg, params['w2_2'], params['b2_2'].reshape(1, DIM),
      params['w3_2'], params['b3_2'].reshape(1, DIM),
      params['ad1_w'], params['ad1_b'].reshape(1, DIM),
      params['ad2_w'], params['ad2_b'].reshape(1, 1))


def kernel(node_type, edge_index, dist, emb, params):
    nt3 = node_type.astype(jnp.int32).reshape(NBLKN, 1, NB)
    emb128 = jnp.pad(emb, ((0, 128 - emb.shape[0]), (0, 0)))
    src = edge_index[0].astype(jnp.int32)
    dst = edge_index[1].astype(jnp.int32)
    src_pad = jnp.pad(src, (0, EPAD - E))
    dst_pad = jnp.pad(dst, (0, EPAD - E), constant_values=N)
    src2d = jnp.concatenate([src_pad, src_pad + N]).reshape(-1, SUBOP)
    dst2d = dst_pad.reshape(-1, SUBOP)
    dist_pad = jnp.pad(dist, ((0, EPAD - E), (0, 0)))
    centers = jnp.asarray(_CENTERS).reshape(1, N_CENTERS)

    w1s = [jnp.stack([params['w1_%d' % l][:, :HALF],
                      params['w1_%d' % l][:, HALF:]]) for l in range(N_CONV)]

    node, nn2 = _emb_call(nt3, emb128, w1s[0])
    hs = _h_call(dist_pad, centers, params)
    agg = None
    for l in range(N_CONV):
        agg = _sc_call(nn2, hs[l], src2d, dst2d)
        if l < N_CONV - 1:
            node, nn2 = _upd_call(node, agg, params, w1s[l + 1], l)
    return _fin_call(node, agg, params)


# speedup vs baseline: 2.8347x; 1.8658x over previous
import functools

import jax
import jax.numpy as jnp
import numpy as np
from jax import lax
from jax.experimental import pallas as pl
from jax.experimental.pallas import tpu as pltpu
from jax.experimental.pallas import tpu_sc as plsc

N = 50000
E = 800000
DIM = 64
HALF = DIM // 2
CUTOFF = 5.0
WIDTH = 0.1
N_CONV = 3
N_CENTERS = 50
_CENTERS = np.linspace(0.0, CUTOFF, N_CENTERS).astype(np.float32)
GAP = float(_CENTERS[1] - _CENTERS[0])

NCORES = 2
NSUB = 16
CH = 128
EPAD = 819200
EPT = EPAD // NSUB
NCHUNK = EPT // CH
NIR = EPAD // CH
NACC = 50048
ROWS_PT = NACC // NSUB

NB = 2000
NBLKN = N // NB
EB = 4096
NEB = -(-E // EB)


def _sp05(x):
    return jnp.where(0.5 * x > 14.0, x,
                     2.0 * jnp.log1p(jnp.exp(jnp.minimum(0.5 * x, 14.0))))


def _ssp(x):
    sp = jnp.where(x > 20.0, x, jnp.log1p(jnp.exp(jnp.minimum(x, 20.0))))
    return sp - np.log(2.0)


def _emb_body(nt_ref, emb_ref, w1h_ref, node_ref, nn_ref):
    nt = nt_ref[0, 0, :]
    oh = (nt[:, None] == lax.broadcasted_iota(jnp.int32, (NB, 128), 1)
          ).astype(jnp.float32)
    node = jax.lax.dot_general(oh, emb_ref[...], (((1,), (0,)), ((), ())),
                               preferred_element_type=jnp.float32)
    node_ref[...] = node
    nn_ref[...] = jnp.dot(node, w1h_ref[0],
                          preferred_element_type=jnp.float32)


def _emb_call(nt3, emb128, w1s):
    return pl.pallas_call(
        _emb_body,
        grid=(NBLKN, NCORES),
        in_specs=[
            pl.BlockSpec((1, 1, NB), lambda i, j: (i, 0, 0)),
            pl.BlockSpec((128, DIM), lambda i, j: (0, 0)),
            pl.BlockSpec((1, DIM, HALF), lambda i, j: (j, 0, 0)),
        ],
        out_specs=[
            pl.BlockSpec((NB, DIM), lambda i, j: (i, 0)),
            pl.BlockSpec((NB, HALF), lambda i, j: (j * NBLKN + i, 0)),
        ],
        out_shape=[
            jax.ShapeDtypeStruct((N, DIM), jnp.float32),
            jax.ShapeDtypeStruct((NCORES * N, HALF), jnp.float32),
        ],
    )(nt3, emb128, w1s)


def _sp05_fast(x):
    return 2.0 * jnp.log1p(jnp.exp(0.5 * x))


def _h_body(dist_ref, cent_ref, cw1, cb1, cw2, cb2, h_ref):
    d = dist_ref[...]
    radial = d - cent_ref[...]
    rbf = jnp.exp((-1.0 / GAP) * radial * radial)
    t = _sp05_fast(jnp.dot(rbf, cw1[...], preferred_element_type=jnp.float32)
                   + cb1[...])
    t = _sp05_fast(jnp.dot(t, cw2[...], preferred_element_type=jnp.float32)
                   + cb2[...])
    h_ref[...] = t


def _h_call(dist, centers, params, l):
    return pl.pallas_call(
        _h_body,
        grid=(NEB,),
        in_specs=[
            pl.BlockSpec((EB, 1), lambda i: (i, 0)),
            pl.BlockSpec((1, N_CENTERS), lambda i: (0, 0)),
            pl.BlockSpec((N_CENTERS, DIM), lambda i: (0, 0)),
            pl.BlockSpec((1, DIM), lambda i: (0, 0)),
            pl.BlockSpec((DIM, DIM), lambda i: (0, 0)),
            pl.BlockSpec((1, DIM), lambda i: (0, 0)),
        ],
        out_specs=pl.BlockSpec((EB, DIM), lambda i: (i, 0)),
        out_shape=jax.ShapeDtypeStruct((EPAD, DIM), jnp.float32),
    )(dist, centers, params['cw1_%d' % l], params['cb1_%d' % l].reshape(1, DIM),
      params['cw2_%d' % l], params['cb2_%d' % l].reshape(1, DIM))


NCH2 = NCHUNK // 2


def _sc_body(nn_ref, h_ref, idx_ref, agg_ref,
             sdA, sdB, rA, rB, hA, hB, acc,
             gsA, gsB, hsA, hsB, ssA, ssB):
    c = lax.axis_index("c")
    s = lax.axis_index("s")

    ibase = 2 * (c * NIR + s * NCHUNK)
    ebase = s * EPT
    hoff = c * HALF

    def hsrc(k):
        return h_ref.at[pl.ds(ebase + k * CH, CH), pl.ds(hoff, HALF)]

    def stage(k, sd):
        pltpu.sync_copy(idx_ref.at[pl.ds(ibase + 2 * k, 2)], sd)

    def fire(k, sd, rb, hb, gs, hs_):
        pltpu.async_copy(hsrc(k), hb, hs_)
        pltpu.async_copy(nn_ref.at[sd.at[0]], rb, gs)

    def wait_gh(k, sd, rb, hb, gs, hs_):
        pltpu.make_async_copy(hsrc(k), hb, hs_).wait()
        pltpu.make_async_copy(nn_ref.at[sd.at[0]], rb, gs).wait()

    def mul(rb, hb):
        def mbody(i, _):
            rb[i, pl.ds(0, 16)] = rb[i, pl.ds(0, 16)] * hb[i, pl.ds(0, 16)]
            rb[i, pl.ds(16, 16)] = rb[i, pl.ds(16, 16)] * hb[i, pl.ds(16, 16)]
            return 0
        lax.fori_loop(0, CH, mbody, 0)

    def fire_sc(sd, rb, ss):
        pltpu.async_copy(rb, acc.at[sd.at[1]], ss, add=True)

    def wait_sc(sd, rb, ss):
        pltpu.make_async_copy(rb, acc.at[sd.at[1]], ss).wait()

    stage(0, sdA)
    fire(0, sdA, rA, hA, gsA, hsA)

    def zbody(i, _):
        hB[i, pl.ds(0, 16)] = jnp.zeros((16,), jnp.float32)
        hB[i, pl.ds(16, 16)] = jnp.zeros((16,), jnp.float32)
        return 0
    lax.fori_loop(0, CH, zbody, 0)
    r0 = s * ROWS_PT
    for m in range(ROWS_PT // CH):
        pltpu.sync_copy(hB, acc.at[pl.ds(r0 + m * CH, CH)])
    rem = ROWS_PT % CH
    if rem:
        pltpu.sync_copy(hB.at[pl.ds(0, rem)],
                        acc.at[pl.ds(r0 + (ROWS_PT // CH) * CH, rem)])
    plsc.subcore_barrier()

    def body(kk, _):
        k0 = 2 * kk
        @pl.when(kk > 0)
        def _():
            wait_sc(sdB, rB, ssB)
        stage(k0 + 1, sdB)
        fire(k0 + 1, sdB, rB, hB, gsB, hsB)
        wait_gh(k0, sdA, rA, hA, gsA, hsA)
        mul(rA, hA)
        fire_sc(sdA, rA, ssA)
        wait_sc(sdA, rA, ssA)
        @pl.when(kk + 1 < NCH2)
        def _():
            stage(k0 + 2, sdA)
            fire(k0 + 2, sdA, rA, hA, gsA, hsA)
        wait_gh(k0 + 1, sdB, rB, hB, gsB, hsB)
        mul(rB, hB)
        fire_sc(sdB, rB, ssB)
        return 0

    lax.fori_loop(0, NCH2, body, 0)
    wait_sc(sdB, rB, ssB)
    plsc.subcore_barrier()

    pltpu.sync_copy(acc.at[pl.ds(r0, ROWS_PT)],
                    agg_ref.at[c, pl.ds(r0, ROWS_PT)])


def _sc_call(nn2, h2, idx2d):
    mesh = plsc.VectorSubcoreMesh(core_axis_name="c", subcore_axis_name="s",
                                  num_cores=NCORES, num_subcores=NSUB)
    return pl.kernel(
        _sc_body,
        out_type=jax.ShapeDtypeStruct((NCORES, NACC, HALF), jnp.float32),
        mesh=mesh,
        compiler_params=pltpu.CompilerParams(use_tc_tiling_on_sc=False),
        scratch_types=[
            pltpu.VMEM((2, CH), jnp.int32),
            pltpu.VMEM((2, CH), jnp.int32),
            pltpu.VMEM((CH, HALF), jnp.float32),
            pltpu.VMEM((CH, HALF), jnp.float32),
            pltpu.VMEM((CH, HALF), jnp.float32),
            pltpu.VMEM((CH, HALF), jnp.float32),
            pltpu.VMEM_SHARED((NACC, HALF), jnp.float32),
            pltpu.SemaphoreType.DMA,
            pltpu.SemaphoreType.DMA,
            pltpu.SemaphoreType.DMA,
            pltpu.SemaphoreType.DMA,
            pltpu.SemaphoreType.DMA,
            pltpu.SemaphoreType.DMA,
        ],
    )(nn2, h2, idx2d)


def _upd_body(node_ref, agglo_ref, agghi_ref, w2_ref, b2_ref, w3_ref, b3_ref,
              w1h_ref, node_out_ref, nn_ref):
    w2 = w2_ref[...]
    pre = (jnp.dot(agglo_ref[0], w2[:HALF, :],
                   preferred_element_type=jnp.float32)
           + jnp.dot(agghi_ref[0], w2[HALF:, :],
                     preferred_element_type=jnp.float32)
           + b2_ref[...])
    a = _sp05(pre)
    node = (node_ref[...]
            + jnp.dot(a, w3_ref[...], preferred_element_type=jnp.float32)
            + b3_ref[...])
    node_out_ref[...] = node
    nn_ref[...] = jnp.dot(node, w1h_ref[0],
                          preferred_element_type=jnp.float32)


def _upd_call(node, agg, params, w1s, l):
    return pl.pallas_call(
        _upd_body,
        grid=(NBLKN, NCORES),
        in_specs=[
            pl.BlockSpec((NB, DIM), lambda i, j: (i, 0)),
            pl.BlockSpec((1, NB, HALF), lambda i, j: (0, i, 0)),
            pl.BlockSpec((1, NB, HALF), lambda i, j: (1, i, 0)),
            pl.BlockSpec((DIM, DIM), lambda i, j: (0, 0)),
            pl.BlockSpec((1, DIM), lambda i, j: (0, 0)),
            pl.BlockSpec((DIM, DIM), lambda i, j: (0, 0)),
            pl.BlockSpec((1, DIM), lambda i, j: (0, 0)),
            pl.BlockSpec((1, DIM, HALF), lambda i, j: (j, 0, 0)),
        ],
        out_specs=[
            pl.BlockSpec((NB, DIM), lambda i, j: (i, 0)),
            pl.BlockSpec((NB, HALF), lambda i, j: (j * NBLKN + i, 0)),
        ],
        out_shape=[
            jax.ShapeDtypeStruct((N, DIM), jnp.float32),
            jax.ShapeDtypeStruct((NCORES * N, HALF), jnp.float32),
        ],
    )(node, agg, agg, params['w2_%d' % l], params['b2_%d' % l].reshape(1, DIM),
      params['w3_%d' % l], params['b3_%d' % l].reshape(1, DIM), w1s)


def _fin_body(node_ref, agglo_ref, agghi_ref, w2_ref, b2_ref, w3_ref, b3_ref,
              ad1w_ref, ad1b_ref, ad2w_ref, ad2b_ref, out_ref):
    i = pl.program_id(0)
    w2 = w2_ref[...]
    pre = (jnp.dot(agglo_ref[0], w2[:HALF, :],
                   preferred_element_type=jnp.float32)
           + jnp.dot(agghi_ref[0], w2[HALF:, :],
                     preferred_element_type=jnp.float32)
           + b2_ref[...])
    a = _sp05(pre)
    node = (node_ref[...]
            + jnp.dot(a, w3_ref[...], preferred_element_type=jnp.float32)
            + b3_ref[...])
    atom = _ssp(jnp.dot(node, ad1w_ref[...],
                        preferred_element_type=jnp.float32) + ad1b_ref[...])
    res = jnp.dot(atom, ad2w_ref[...], preferred_element_type=jnp.float32)
    part = (jnp.sum(res) + NB * ad2b_ref[0, 0]) * (1.0 / N)

    @pl.when(i == 0)
    def _():
        out_ref[...] = jnp.zeros((1, 1), jnp.float32)
    out_ref[...] = out_ref[...] + part


def _fin_call(node, agg, params):
    return pl.pallas_call(
        _fin_body,
        grid=(NBLKN,),
        in_specs=[
            pl.BlockSpec((NB, DIM), lambda i: (i, 0)),
            pl.BlockSpec((1, NB, HALF), lambda i: (0, i, 0)),
            pl.BlockSpec((1, NB, HALF), lambda i: (1, i, 0)),
            pl.BlockSpec((DIM, DIM), lambda i: (0, 0)),
            pl.BlockSpec((1, DIM), lambda i: (0, 0)),
            pl.BlockSpec((DIM, DIM), lambda i: (0, 0)),
            pl.BlockSpec((1, DIM), lambda i: (0, 0)),
            pl.BlockSpec((DIM, DIM), lambda i: (0, 0)),
            pl.BlockSpec((1, DIM), lambda i: (0, 0)),
            pl.BlockSpec((DIM, 1), lambda i: (0, 0)),
            pl.BlockSpec((1, 1), lambda i: (0, 0)),
        ],
        out_specs=pl.BlockSpec((1, 1), lambda i: (0, 0)),
        out_shape=jax.ShapeDtypeStruct((1, 1), jnp.float32),
    )(node, agg, agg, params['w2_2'], params['b2_2'].reshape(1, DIM),
      params['w3_2'], params['b3_2'].reshape(1, DIM),
      params['ad1_w'], params['ad1_b'].reshape(1, DIM),
      params['ad2_w'], params['ad2_b'].reshape(1, 1))


def kernel(node_type, edge_index, dist, emb, params):
    nt3 = node_type.astype(jnp.int32).reshape(NBLKN, 1, NB)
    emb128 = jnp.pad(emb, ((0, 128 - emb.shape[0]), (0, 0)))
    src = edge_index[0].astype(jnp.int32)
    dst = edge_index[1].astype(jnp.int32)
    src_pad = jnp.pad(src, (0, EPAD - E))
    dst_pad = jnp.pad(dst, (0, EPAD - E), constant_values=N)
    s2 = jnp.stack([src_pad, src_pad + N]).reshape(NCORES, NIR, CH)
    d2 = jnp.broadcast_to(dst_pad.reshape(1, NIR, CH), (NCORES, NIR, CH))
    idx2d = jnp.stack([s2, d2], axis=2).reshape(-1, CH)
    centers = jnp.asarray(_CENTERS).reshape(1, N_CENTERS)

    w1s = [jnp.stack([params['w1_%d' % l][:, :HALF],
                      params['w1_%d' % l][:, HALF:]]) for l in range(N_CONV)]

    node, nn2 = _emb_call(nt3, emb128, w1s[0])
    agg = None
    for l in range(N_CONV):
        h = _h_call(dist, centers, params, l)
        agg = _sc_call(nn2, h, idx2d)
        if l < N_CONV - 1:
            node, nn2 = _upd_call(node, agg, params, w1s[l + 1], l)
    return _fin_call(node, agg, params)


# speedup vs baseline: 3.2247x; 1.1376x over previous
import functools

import jax
import jax.numpy as jnp
import numpy as np
from jax import lax
from jax.experimental import pallas as pl
from jax.experimental.pallas import tpu as pltpu
from jax.experimental.pallas import tpu_sc as plsc

N = 50000
E = 800000
DIM = 64
HALF = DIM // 2
CUTOFF = 5.0
WIDTH = 0.1
N_CONV = 3
N_CENTERS = 50
_CENTERS = np.linspace(0.0, CUTOFF, N_CENTERS).astype(np.float32)
GAP = float(_CENTERS[1] - _CENTERS[0])

NCORES = 2
NSUB = 16
CH = 128
EPAD = 819200
EPT = EPAD // NSUB
NCHUNK = EPT // CH
NIR = EPAD // CH
NACC = 50048
ROWS_PT = NACC // NSUB

NB = 2000
NBLKN = N // NB
EB = 4096
NEB = -(-E // EB)


def _sp05(x):
    return jnp.where(0.5 * x > 14.0, x,
                     2.0 * jnp.log1p(jnp.exp(jnp.minimum(0.5 * x, 14.0))))


def _ssp(x):
    sp = jnp.where(x > 20.0, x, jnp.log1p(jnp.exp(jnp.minimum(x, 20.0))))
    return sp - np.log(2.0)


def _emb_body(nt_ref, emb_ref, w1h_ref, node_ref, nn_ref):
    nt = nt_ref[0, 0, :]
    oh = (nt[:, None] == lax.broadcasted_iota(jnp.int32, (NB, 128), 1)
          ).astype(jnp.float32)
    node = jax.lax.dot_general(oh, emb_ref[...], (((1,), (0,)), ((), ())),
                               preferred_element_type=jnp.float32)
    node_ref[...] = node
    nn_ref[...] = jnp.dot(node, w1h_ref[0],
                          preferred_element_type=jnp.float32)


def _emb_call(nt3, emb128, w1s):
    return pl.pallas_call(
        _emb_body,
        grid=(NBLKN, NCORES),
        in_specs=[
            pl.BlockSpec((1, 1, NB), lambda i, j: (i, 0, 0)),
            pl.BlockSpec((128, DIM), lambda i, j: (0, 0)),
            pl.BlockSpec((1, DIM, HALF), lambda i, j: (j, 0, 0)),
        ],
        out_specs=[
            pl.BlockSpec((NB, DIM), lambda i, j: (i, 0)),
            pl.BlockSpec((NB, HALF), lambda i, j: (j * NBLKN + i, 0)),
        ],
        out_shape=[
            jax.ShapeDtypeStruct((N, DIM), jnp.float32),
            jax.ShapeDtypeStruct((NCORES * N, HALF), jnp.float32),
        ],
    )(nt3, emb128, w1s)


def _sp05_fast(x):
    return 2.0 * jnp.log1p(jnp.exp(0.5 * x))


EB2 = EB // 2


def _h_body(d2_ref, sel_ref, cent2_ref, cw1d, cb1d, cw2d, cb2d, h_ref):
    dw = jnp.dot(d2_ref[...], sel_ref[...],
                 preferred_element_type=jnp.float32)
    radial = dw - cent2_ref[...]
    rbf = jnp.exp((-1.0 / GAP) * radial * radial)
    t = _sp05_fast(jnp.dot(rbf, cw1d[...], preferred_element_type=jnp.float32)
                   + cb1d[...])
    t = _sp05_fast(jnp.dot(t, cw2d[...], preferred_element_type=jnp.float32)
                   + cb2d[...])
    h_ref[...] = t


def _h_call(dist2, sel, cent2, cw1d, cb1d, cw2d, cb2d):
    return pl.pallas_call(
        _h_body,
        grid=(NEB,),
        in_specs=[
            pl.BlockSpec((EB2, 2), lambda i: (i, 0)),
            pl.BlockSpec((2, 2 * N_CENTERS), lambda i: (0, 0)),
            pl.BlockSpec((1, 2 * N_CENTERS), lambda i: (0, 0)),
            pl.BlockSpec((2 * N_CENTERS, 2 * DIM), lambda i: (0, 0)),
            pl.BlockSpec((1, 2 * DIM), lambda i: (0, 0)),
            pl.BlockSpec((2 * DIM, 2 * DIM), lambda i: (0, 0)),
            pl.BlockSpec((1, 2 * DIM), lambda i: (0, 0)),
        ],
        out_specs=pl.BlockSpec((EB2, 2 * DIM), lambda i: (i, 0)),
        out_shape=jax.ShapeDtypeStruct((EPAD // 2, 2 * DIM), jnp.float32),
    )(dist2, sel, cent2, cw1d, cb1d, cw2d, cb2d)


NCH2 = NCHUNK // 2


def _sc_body(nn_ref, h_ref, idx_ref, agg_ref,
             sdA, sdB, rA, rB, hA, hB, acc,
             gsA, gsB, hsA, hsB, ssA, ssB):
    c = lax.axis_index("c")
    s = lax.axis_index("s")

    ibase = 2 * (c * NIR + s * NCHUNK)
    ebase2 = s * (EPT // 2)
    CH2 = CH // 2

    def hsrc(k):
        return h_ref.at[pl.ds(ebase2 + k * CH2, CH2)]

    def stage(k, sd):
        pltpu.sync_copy(idx_ref.at[pl.ds(ibase + 2 * k, 2)], sd)

    def fire(k, sd, rb, hb, gs, hs_):
        pltpu.async_copy(hsrc(k), hb, hs_)
        pltpu.async_copy(nn_ref.at[sd.at[0]], rb, gs)

    def wait_gh(k, sd, rb, hb, gs, hs_):
        pltpu.make_async_copy(hsrc(k), hb, hs_).wait()
        pltpu.make_async_copy(nn_ref.at[sd.at[0]], rb, gs).wait()

    def mul(rb, hb):
        def mk(co):
            def mbody(i, _):
                r = 2 * i
                rb[r, pl.ds(0, 16)] = (rb[r, pl.ds(0, 16)]
                                       * hb[i, pl.ds(co, 16)])
                rb[r, pl.ds(16, 16)] = (rb[r, pl.ds(16, 16)]
                                        * hb[i, pl.ds(co + 16, 16)])
                rb[r + 1, pl.ds(0, 16)] = (rb[r + 1, pl.ds(0, 16)]
                                           * hb[i, pl.ds(co + 64, 16)])
                rb[r + 1, pl.ds(16, 16)] = (rb[r + 1, pl.ds(16, 16)]
                                            * hb[i, pl.ds(co + 80, 16)])
                return 0
            return mbody

        @pl.when(c == 0)
        def _():
            lax.fori_loop(0, CH2, mk(0), 0)

        @pl.when(c == 1)
        def _():
            lax.fori_loop(0, CH2, mk(HALF), 0)

    def fire_sc(sd, rb, ss):
        pltpu.async_copy(rb, acc.at[sd.at[1]], ss, add=True)

    def wait_sc(sd, rb, ss):
        pltpu.make_async_copy(rb, acc.at[sd.at[1]], ss).wait()

    stage(0, sdA)
    fire(0, sdA, rA, hA, gsA, hsA)

    def zbody(i, _):
        rB[i, pl.ds(0, 16)] = jnp.zeros((16,), jnp.float32)
        rB[i, pl.ds(16, 16)] = jnp.zeros((16,), jnp.float32)
        return 0
    lax.fori_loop(0, CH, zbody, 0)
    r0 = s * ROWS_PT
    for m in range(ROWS_PT // CH):
        pltpu.sync_copy(rB, acc.at[pl.ds(r0 + m * CH, CH)])
    rem = ROWS_PT % CH
    if rem:
        pltpu.sync_copy(rB.at[pl.ds(0, rem)],
                        acc.at[pl.ds(r0 + (ROWS_PT // CH) * CH, rem)])
    plsc.subcore_barrier()

    def body(kk, _):
        k0 = 2 * kk
        @pl.when(kk > 0)
        def _():
            wait_sc(sdB, rB, ssB)
        stage(k0 + 1, sdB)
        fire(k0 + 1, sdB, rB, hB, gsB, hsB)
        wait_gh(k0, sdA, rA, hA, gsA, hsA)
        mul(rA, hA)
        fire_sc(sdA, rA, ssA)
        wait_sc(sdA, rA, ssA)
        @pl.when(kk + 1 < NCH2)
        def _():
            stage(k0 + 2, sdA)
            fire(k0 + 2, sdA, rA, hA, gsA, hsA)
        wait_gh(k0 + 1, sdB, rB, hB, gsB, hsB)
        mul(rB, hB)
        fire_sc(sdB, rB, ssB)
        return 0

    lax.fori_loop(0, NCH2, body, 0)
    wait_sc(sdB, rB, ssB)
    plsc.subcore_barrier()

    pltpu.sync_copy(acc.at[pl.ds(r0, ROWS_PT)],
                    agg_ref.at[c, pl.ds(r0, ROWS_PT)])


def _sc_call(nn2, h2, idx2d):
    mesh = plsc.VectorSubcoreMesh(core_axis_name="c", subcore_axis_name="s",
                                  num_cores=NCORES, num_subcores=NSUB)
    return pl.kernel(
        _sc_body,
        out_type=jax.ShapeDtypeStruct((NCORES, NACC, HALF), jnp.float32),
        mesh=mesh,
        compiler_params=pltpu.CompilerParams(use_tc_tiling_on_sc=False),
        scratch_types=[
            pltpu.VMEM((2, CH), jnp.int32),
            pltpu.VMEM((2, CH), jnp.int32),
            pltpu.VMEM((CH, HALF), jnp.float32),
            pltpu.VMEM((CH, HALF), jnp.float32),
            pltpu.VMEM((CH // 2, 2 * DIM), jnp.float32),
            pltpu.VMEM((CH // 2, 2 * DIM), jnp.float32),
            pltpu.VMEM_SHARED((NACC, HALF), jnp.float32),
            pltpu.SemaphoreType.DMA,
            pltpu.SemaphoreType.DMA,
            pltpu.SemaphoreType.DMA,
            pltpu.SemaphoreType.DMA,
            pltpu.SemaphoreType.DMA,
            pltpu.SemaphoreType.DMA,
        ],
    )(nn2, h2, idx2d)


def _upd_body(node_ref, agglo_ref, agghi_ref, w2_ref, b2_ref, w3_ref, b3_ref,
              w1h_ref, node_out_ref, nn_ref):
    w2 = w2_ref[...]
    pre = (jnp.dot(agglo_ref[0], w2[:HALF, :],
                   preferred_element_type=jnp.float32)
           + jnp.dot(agghi_ref[0], w2[HALF:, :],
                     preferred_element_type=jnp.float32)
           + b2_ref[...])
    a = _sp05(pre)
    node = (node_ref[...]
            + jnp.dot(a, w3_ref[...], preferred_element_type=jnp.float32)
            + b3_ref[...])
    node_out_ref[...] = node
    nn_ref[...] = jnp.dot(node, w1h_ref[0],
                          preferred_element_type=jnp.float32)


def _upd_call(node, agg, params, w1s, l):
    return pl.pallas_call(
        _upd_body,
        grid=(NBLKN, NCORES),
        in_specs=[
            pl.BlockSpec((NB, DIM), lambda i, j: (i, 0)),
            pl.BlockSpec((1, NB, HALF), lambda i, j: (0, i, 0)),
            pl.BlockSpec((1, NB, HALF), lambda i, j: (1, i, 0)),
            pl.BlockSpec((DIM, DIM), lambda i, j: (0, 0)),
            pl.BlockSpec((1, DIM), lambda i, j: (0, 0)),
            pl.BlockSpec((DIM, DIM), lambda i, j: (0, 0)),
            pl.BlockSpec((1, DIM), lambda i, j: (0, 0)),
            pl.BlockSpec((1, DIM, HALF), lambda i, j: (j, 0, 0)),
        ],
        out_specs=[
            pl.BlockSpec((NB, DIM), lambda i, j: (i, 0)),
            pl.BlockSpec((NB, HALF), lambda i, j: (j * NBLKN + i, 0)),
        ],
        out_shape=[
            jax.ShapeDtypeStruct((N, DIM), jnp.float32),
            jax.ShapeDtypeStruct((NCORES * N, HALF), jnp.float32),
        ],
    )(node, agg, agg, params['w2_%d' % l], params['b2_%d' % l].reshape(1, DIM),
      params['w3_%d' % l], params['b3_%d' % l].reshape(1, DIM), w1s)


def _fin_body(node_ref, agglo_ref, agghi_ref, w2_ref, b2_ref, w3_ref, b3_ref,
              ad1w_ref, ad1b_ref, ad2w_ref, ad2b_ref, out_ref):
    i = pl.program_id(0)
    w2 = w2_ref[...]
    pre = (jnp.dot(agglo_ref[0], w2[:HALF, :],
                   preferred_element_type=jnp.float32)
           + jnp.dot(agghi_ref[0], w2[HALF:, :],
                     preferred_element_type=jnp.float32)
           + b2_ref[...])
    a = _sp05(pre)
    node = (node_ref[...]
            + jnp.dot(a, w3_ref[...], preferred_element_type=jnp.float32)
            + b3_ref[...])
    atom = _ssp(jnp.dot(node, ad1w_ref[...],
                        preferred_element_type=jnp.float32) + ad1b_ref[...])
    res = jnp.dot(atom, ad2w_ref[...], preferred_element_type=jnp.float32)
    part = (jnp.sum(res) + NB * ad2b_ref[0, 0]) * (1.0 / N)

    @pl.when(i == 0)
    def _():
        out_ref[...] = jnp.zeros((1, 1), jnp.float32)
    out_ref[...] = out_ref[...] + part


def _fin_call(node, agg, params):
    return pl.pallas_call(
        _fin_body,
        grid=(NBLKN,),
        in_specs=[
            pl.BlockSpec((NB, DIM), lambda i: (i, 0)),
            pl.BlockSpec((1, NB, HALF), lambda i: (0, i, 0)),
            pl.BlockSpec((1, NB, HALF), lambda i: (1, i, 0)),
            pl.BlockSpec((DIM, DIM), lambda i: (0, 0)),
            pl.BlockSpec((1, DIM), lambda i: (0, 0)),
            pl.BlockSpec((DIM, DIM), lambda i: (0, 0)),
            pl.BlockSpec((1, DIM), lambda i: (0, 0)),
            pl.BlockSpec((DIM, DIM), lambda i: (0, 0)),
            pl.BlockSpec((1, DIM), lambda i: (0, 0)),
            pl.BlockSpec((DIM, 1), lambda i: (0, 0)),
            pl.BlockSpec((1, 1), lambda i: (0, 0)),
        ],
        out_specs=pl.BlockSpec((1, 1), lambda i: (0, 0)),
        out_shape=jax.ShapeDtypeStruct((1, 1), jnp.float32),
    )(node, agg, agg, params['w2_2'], params['b2_2'].reshape(1, DIM),
      params['w3_2'], params['b3_2'].reshape(1, DIM),
      params['ad1_w'], params['ad1_b'].reshape(1, DIM),
      params['ad2_w'], params['ad2_b'].reshape(1, 1))


def kernel(node_type, edge_index, dist, emb, params):
    nt3 = node_type.astype(jnp.int32).reshape(NBLKN, 1, NB)
    emb128 = jnp.pad(emb, ((0, 128 - emb.shape[0]), (0, 0)))
    src = edge_index[0].astype(jnp.int32)
    dst = edge_index[1].astype(jnp.int32)
    src_pad = jnp.pad(src, (0, EPAD - E))
    dst_pad = jnp.pad(dst, (0, EPAD - E), constant_values=N)
    s2 = jnp.stack([src_pad, src_pad + N]).reshape(NCORES, NIR, CH)
    d2 = jnp.broadcast_to(dst_pad.reshape(1, NIR, CH), (NCORES, NIR, CH))
    idx2d = jnp.stack([s2, d2], axis=2).reshape(-1, CH)
    dist2 = dist.reshape(-1, 2)
    sel = jnp.kron(jnp.eye(2, dtype=jnp.float32),
                   jnp.ones((1, N_CENTERS), jnp.float32))
    cent2 = jnp.tile(jnp.asarray(_CENTERS), 2).reshape(1, 2 * N_CENTERS)
    z = jnp.zeros
    hargs = []
    for l in range(N_CONV):
        cw1 = params['cw1_%d' % l]
        cw2 = params['cw2_%d' % l]
        cw1d = jnp.concatenate([
            jnp.concatenate([cw1, z(cw1.shape, jnp.float32)], axis=1),
            jnp.concatenate([z(cw1.shape, jnp.float32), cw1], axis=1)],
            axis=0)
        cw2d = jnp.concatenate([
            jnp.concatenate([cw2, z(cw2.shape, jnp.float32)], axis=1),
            jnp.concatenate([z(cw2.shape, jnp.float32), cw2], axis=1)],
            axis=0)
        cb1d = jnp.tile(params['cb1_%d' % l], 2).reshape(1, 2 * DIM)
        cb2d = jnp.tile(params['cb2_%d' % l], 2).reshape(1, 2 * DIM)
        hargs.append((cw1d, cb1d, cw2d, cb2d))

    w1s = [jnp.stack([params['w1_%d' % l][:, :HALF],
                      params['w1_%d' % l][:, HALF:]]) for l in range(N_CONV)]

    node, nn2 = _emb_call(nt3, emb128, w1s[0])
    agg = None
    for l in range(N_CONV):
        h = _h_call(dist2, sel, cent2, *hargs[l])
        agg = _sc_call(nn2, h, idx2d)
        if l < N_CONV - 1:
            node, nn2 = _upd_call(node, agg, params, w1s[l + 1], l)
    return _fin_call(node, agg, params)
